# baseline (speedup 1.0000x reference)
import functools

import jax
import jax.numpy as jnp
from jax import lax
from jax.experimental import pallas as pl
from jax.experimental.pallas import tpu as pltpu
from jax.experimental.pallas import tpu_sc as plsc

DELTA_V = 0.5
DELTA_D = 1.5
ALPHA = 1.0
BETA = 1.0
GAMMA = 0.001

BB = 4
EE = 16
PP = 512 * 512
NI = 17

NW = 32
TPB = NW // BB
PIX_PER_TILE = PP // TPB
CHUNK = 2048
NGROUP = CHUNK // 16
NCHUNK = PIX_PER_TILE // CHUNK

LSTRIDE = NI * EE + 1

PREC = NI * EE + 3 * 16
SROW = 512
OROWS = 40

_mesh = plsc.VectorSubcoreMesh(core_axis_name="c", subcore_axis_name="s")


def _vsqrt(x):
    i = lax.bitcast_convert_type(x, jnp.int32)
    r = lax.bitcast_convert_type(jnp.int32(0x5F3759DF) - (i >> 1), jnp.float32)
    for _ in range(3):
        r = r * (1.5 - 0.5 * x * r * r)
    return x * r


def _zero_ref(ref, nwords):
    zf = jnp.zeros((16,), jnp.float32)

    def body(j, _):
        ref[pl.ds(j * 16, 16)] = zf
        return 0

    lax.fori_loop(0, nwords // 16, body, 0)


def _pixel_pipeline(emb, mask, b, tile_base, embbuf, maskbuf, sems, process, carry0):
    def start(slot, cbase):
        for c in range(EE):
            pltpu.async_copy(
                emb.at[b, c, pl.ds(cbase, CHUNK)], embbuf.at[slot, c], sems[slot]
            )
        pltpu.async_copy(mask.at[b, pl.ds(cbase, CHUNK)], maskbuf.at[slot], sems[slot])

    def drain(slot):
        for c in range(EE):
            pltpu.make_async_copy(
                emb.at[0, c, pl.ds(0, CHUNK)], embbuf.at[slot, c], sems[slot]
            ).wait()
        pltpu.make_async_copy(
            mask.at[0, pl.ds(0, CHUNK)], maskbuf.at[slot], sems[slot]
        ).wait()

    start(0, tile_base)

    def pair(k2, cy):
        base0 = tile_base + (2 * k2) * CHUNK
        start(1, base0 + CHUNK)
        drain(0)
        cy = process(0, cy)
        start(0, jnp.minimum(base0 + 2 * CHUNK, PP - CHUNK))
        drain(1)
        cy = process(1, cy)
        return cy

    carry = lax.fori_loop(0, NCHUNK // 2, pair, carry0)
    drain(0)
    return carry


@functools.partial(
    pl.kernel,
    out_type=jax.ShapeDtypeStruct((NW, OROWS * 16), jnp.float32),
    mesh=_mesh,
    scratch_types=[
        pltpu.VMEM((2, EE, CHUNK), jnp.float32),
        pltpu.VMEM((2, CHUNK), jnp.int32),
        pltpu.VMEM((16 * LSTRIDE,), jnp.float32),
        pltpu.VMEM((16 * NI,), jnp.float32),
        pltpu.VMEM((SROW,), jnp.float32),
        pltpu.VMEM_SHARED((16, SROW), jnp.float32),
        pltpu.VMEM((TPB, SROW), jnp.float32),
        pltpu.VMEM((PREC,), jnp.float32),
        pltpu.VMEM((NI * EE,), jnp.float32),
        pltpu.VMEM((OROWS * 16,), jnp.float32),
        pltpu.SemaphoreType.DMA,
        pltpu.SemaphoreType.DMA,
    ],
    compiler_params=pltpu.CompilerParams(needs_layout_passes=False),
)
def _main(
    emb, mask, out,
    embbuf, maskbuf, lsums, lcnt, pbuf, shared, mates, macc, meansbuf, cbuf,
    sem0, sem1,
):
    cid = lax.axis_index("c")
    sid = lax.axis_index("s")
    b = cid * 2 + sid // 8
    tile_base = (sid % 8) * PIX_PER_TILE
    orow = cid * 16 + sid

    iota = lax.iota(jnp.int32, 16)
    lane_s = iota * LSTRIDE
    lane_c = iota * NI
    ones = jnp.ones((16,), jnp.float32)
    zf = jnp.zeros((16,), jnp.float32)

    _zero_ref(lsums, 16 * LSTRIDE - 16)
    lsums[pl.ds(16 * LSTRIDE - 16, 16)] = zf
    _zero_ref(lcnt, 16 * NI)

    def p1(slot, mv):
        def body(g, mvi):
            off = g * 16
            m = maskbuf[slot, pl.ds(off, 16)]
            plsc.addupdate_scatter(lcnt, [lane_c + m], ones)
            basei = lane_s + (m << 4)
            for c in range(EE):
                v = embbuf[slot, c, pl.ds(off, 16)]
                plsc.addupdate_scatter(lsums, [basei + c], v)
            return jnp.maximum(mvi, m)

        return plsc.parallel_loop(0, NGROUP, 1, unroll=4, carry=mv)(body)

    maxv = _pixel_pipeline(
        emb, mask, b, tile_base, embbuf, maskbuf, (sem0, sem1), p1,
        jnp.zeros((16,), jnp.int32),
    )

    def sum_row(m, _):
        acc = zf
        for l in range(16):
            acc = acc + plsc.load_gather(lsums, [iota + (l * LSTRIDE) + m * EE])
        pbuf[pl.ds(m * 16, 16)] = acc
        return 0

    lax.fori_loop(0, NI, sum_row, 0)
    comp = zf
    for l in range(16):
        comp = comp + plsc.load_gather(lcnt, [iota + l * NI])
    pbuf[pl.ds(NI * EE, 16)] = comp
    v16 = plsc.load_gather(lcnt, [lane_c + 16])
    pbuf[pl.ds(NI * EE + 16, 16)] = jnp.full((16,), jnp.sum(v16), jnp.float32)
    nmax = jnp.max(maxv).astype(jnp.float32)
    pbuf[pl.ds(NI * EE + 32, 16)] = jnp.full((16,), nmax, jnp.float32)

    pltpu.sync_copy(pbuf, shared.at[sid])
    plsc.subcore_barrier()
    g0 = (sid // 8) * TPB
    for t in range(TPB):
        pltpu.sync_copy(shared.at[g0 + t], mates.at[t])

    def mred(j, _):
        off = j * 16
        a = mates[0, pl.ds(off, 16)]
        for t in range(1, TPB):
            a = a + mates[t, pl.ds(off, 16)]
        macc[pl.ds(off, 16)] = a
        return 0

    lax.fori_loop(0, PREC // 16 - 1, mred, 0)
    a = mates[0, pl.ds(PREC - 16, 16)]
    for t in range(1, TPB):
        a = jnp.maximum(a, mates[t, pl.ds(PREC - 16, 16)])
    macc[pl.ds(PREC - 16, 16)] = a

    def mean_row(m, _):
        cvec = plsc.load_gather(macc, [jnp.full((16,), NI * EE, jnp.int32) + m])
        row = macc[pl.ds(m * 16, 16)]
        meansbuf[pl.ds(m * 16, 16)] = jnp.where(
            cvec > 0.0, row / jnp.maximum(cvec, 1.0), 0.0
        )
        return 0

    lax.fori_loop(0, NI, mean_row, 0)

    _zero_ref(lcnt, 16 * NI)

    def p2(slot, cy):
        def body(g):
            off = g * 16
            m = maskbuf[slot, pl.ds(off, 16)]
            base16 = m << 4
            accs = [zf for _ in range(4)]
            for c in range(EE):
                v = embbuf[slot, c, pl.ds(off, 16)]
                mu = plsc.load_gather(meansbuf, [base16 + c])
                dlt = v - mu
                accs[c % 4] = accs[c % 4] + dlt * dlt
            acc = (accs[0] + accs[1]) + (accs[2] + accs[3])
            dist = _vsqrt(acc)
            h = jnp.maximum(dist - DELTA_V, 0.0)
            plsc.addupdate_scatter(lcnt, [lane_c + m], h * h)

        plsc.parallel_loop(0, NGROUP, 1, unroll=4)(body)
        return cy

    _pixel_pipeline(emb, mask, b, tile_base, embbuf, maskbuf, (sem0, sem1), p2, 0)

    def var_row(m, _):
        v = plsc.load_gather(lcnt, [lane_c + m])
        cbuf[pl.ds(m * 16, 16)] = jnp.full((16,), jnp.sum(v), jnp.float32)
        return 0

    lax.fori_loop(0, NI, var_row, 0)

    nvec = macc[pl.ds(PREC - 16, 16)]
    idsf = (iota + 1).astype(jnp.float32)
    mjs = tuple(
        plsc.load_gather(meansbuf, [(iota + 1) * 16 + c]) for c in range(EE)
    )
    validj = jnp.where(idsf <= nvec, 1.0, 0.0)

    def pair_i(i, carry):
        psum, pcnt = carry
        base = (i + 1) * 16
        accs = [zf for _ in range(4)]
        for c in range(EE):
            mi = plsc.load_gather(meansbuf, [jnp.full((16,), base, jnp.int32) + c])
            d = mjs[c] - mi
            accs[c % 4] = accs[c % 4] + d * d
        acc = (accs[0] + accs[1]) + (accs[2] + accs[3])
        dist = _vsqrt(acc)
        h = jnp.maximum(2.0 * DELTA_D - dist, 0.0)
        ifl = (i + 1).astype(jnp.float32)
        pv = jnp.where((idsf > ifl) & (ifl <= nvec), validj, 0.0)
        return psum + h * h * pv, pcnt + pv

    psum, pcnt = lax.fori_loop(0, 16, pair_i, (zf, zf))
    tot = jnp.full((16,), jnp.sum(psum), jnp.float32)
    npair = jnp.full((16,), jnp.sum(pcnt), jnp.float32)
    dist_row = jnp.where(nvec > 1.0, tot / jnp.maximum(npair, 1.0), 0.0)
    cbuf[pl.ds(NI * 16, 16)] = dist_row

    nsq = zf
    for c in range(EE):
        nsq = nsq + mjs[c] * mjs[c]
    nr = _vsqrt(nsq)
    regsum = jnp.full((16,), jnp.sum(nr * validj), jnp.float32)
    reg_row = jnp.where(nvec > 0.0, regsum / jnp.maximum(nvec, 1.0), 0.0)
    cbuf[pl.ds((NI + 1) * 16, 16)] = reg_row

    def cnt_row(m, _):
        cvec = plsc.load_gather(macc, [jnp.full((16,), NI * EE, jnp.int32) + m])
        cbuf[pl.ds((NI + 2) * 16 + m * 16, 16)] = cvec
        return 0

    lax.fori_loop(0, NI, cnt_row, 0)
    cbuf[pl.ds((2 * NI + 2) * 16, 16)] = nvec
    for r in range(2 * NI + 3, OROWS):
        cbuf[pl.ds(r * 16, 16)] = zf

    pltpu.sync_copy(cbuf, out.at[orow])


def _fin_body(v_ref, out_ref):
    tv = jnp.float32(0.0)
    td = jnp.float32(0.0)
    tr = jnp.float32(0.0)
    valid = jnp.float32(0.0)
    for b in range(BB):
        V = v_ref[TPB * b]
        for t in range(1, TPB):
            V = V + v_ref[TPB * b + t]
        Vm = V[1:NI]
        lead = v_ref[TPB * b]
        C = lead[NI + 3 : 2 * NI + 2]
        n_rep = lead[2 * NI + 2 : 2 * NI + 3]
        lm = Vm / jnp.maximum(C, 1.0)
        pres = jnp.where(C > 0.0, 1.0, 0.0)
        npres = jnp.sum(pres[:, 0:1])
        vsum = jnp.sum(lm[:, 0:1])
        v_b = jnp.where(npres > 0.0, vsum / jnp.maximum(npres, 1.0), 0.0)
        n_sc = jnp.max(n_rep)
        has = jnp.where(n_sc > 0.0, 1.0, 0.0)
        tv = tv + has * v_b
        td = td + has * jnp.max(lead[NI : NI + 1])
        tr = tr + has * jnp.max(lead[NI + 1 : NI + 2])
        valid = valid + has
    vf = jnp.maximum(valid, 1.0)
    tv = jnp.where(valid > 0.0, tv / vf, tv)
    td = jnp.where(valid > 0.0, td / vf, td)
    tr = jnp.where(valid > 0.0, tr / vf, tr)
    loss = ALPHA * tv + BETA * td + GAMMA * tr
    out_ref[0:1, :] = jnp.full((1, 128), loss, jnp.float32)
    out_ref[1:2, :] = jnp.full((1, 128), tv, jnp.float32)
    out_ref[2:3, :] = jnp.full((1, 128), td, jnp.float32)
    out_ref[3:4, :] = jnp.full((1, 128), tr, jnp.float32)
    out_ref[4:8, :] = jnp.zeros((4, 128), jnp.float32)


_fin = pl.pallas_call(
    _fin_body,
    out_shape=jax.ShapeDtypeStruct((8, 128), jnp.float32),
)


@jax.jit
def kernel(embeddings, instance_masks):
    emb = embeddings.reshape(BB, EE, PP)
    mask = instance_masks.reshape(BB, PP).astype(jnp.int32)
    part = _main(emb, mask)
    out = _fin(part.reshape(NW, OROWS, 16))
    return out[0, 0], out[1, 0], out[2, 0], out[3, 0]


# speedup vs baseline: 1.0499x; 1.0499x over previous
import functools

import jax
import jax.numpy as jnp
from jax import lax
from jax.experimental import pallas as pl
from jax.experimental.pallas import tpu as pltpu
from jax.experimental.pallas import tpu_sc as plsc

DELTA_V = 0.5
DELTA_D = 1.5
ALPHA = 1.0
BETA = 1.0
GAMMA = 0.001

BB = 4
EE = 16
PP = 512 * 512
NI = 17

NW = 32
TPB = NW // BB
PIX_PER_TILE = PP // TPB
CHUNK = 2048
NGROUP = CHUNK // 16
NCHUNK = PIX_PER_TILE // CHUNK

LSTRIDE = NI * EE + 1

PREC = NI * EE + 3 * 16
SROW = 512
OROWS = 40

_mesh = plsc.VectorSubcoreMesh(core_axis_name="c", subcore_axis_name="s")


def _vsqrt(x):
    i = lax.bitcast_convert_type(x, jnp.int32)
    r = lax.bitcast_convert_type(jnp.int32(0x5F3759DF) - (i >> 1), jnp.float32)
    for _ in range(3):
        r = r * (1.5 - 0.5 * x * r * r)
    return x * r


def _zero_ref(ref, nwords):
    zf = jnp.zeros((16,), jnp.float32)

    def body(j, _):
        ref[pl.ds(j * 16, 16)] = zf
        return 0

    lax.fori_loop(0, nwords // 16, body, 0)


def _pixel_pipeline(emb, mask, b, tile_base, embbuf, maskbuf, sems, process, carry0):
    def start(slot, cbase):
        for c in range(EE):
            pltpu.async_copy(
                emb.at[b, c, pl.ds(cbase, CHUNK)], embbuf.at[slot, c], sems[slot]
            )
        pltpu.async_copy(mask.at[b, pl.ds(cbase, CHUNK)], maskbuf.at[slot], sems[slot])

    def drain(slot):
        for c in range(EE):
            pltpu.make_async_copy(
                emb.at[0, c, pl.ds(0, CHUNK)], embbuf.at[slot, c], sems[slot]
            ).wait()
        pltpu.make_async_copy(
            mask.at[0, pl.ds(0, CHUNK)], maskbuf.at[slot], sems[slot]
        ).wait()

    start(0, tile_base)

    def pair(k2, cy):
        base0 = tile_base + (2 * k2) * CHUNK
        start(1, base0 + CHUNK)
        drain(0)
        cy = process(0, cy)
        start(0, jnp.minimum(base0 + 2 * CHUNK, PP - CHUNK))
        drain(1)
        cy = process(1, cy)
        return cy

    carry = lax.fori_loop(0, NCHUNK // 2, pair, carry0)
    drain(0)
    return carry


@functools.partial(
    pl.kernel,
    out_type=jax.ShapeDtypeStruct((NW, OROWS * 16), jnp.float32),
    mesh=_mesh,
    scratch_types=[
        pltpu.VMEM((2, EE, CHUNK), jnp.float32),
        pltpu.VMEM((2, CHUNK), jnp.int32),
        pltpu.VMEM((16 * LSTRIDE,), jnp.float32),
        pltpu.VMEM((16 * NI,), jnp.float32),
        pltpu.VMEM((SROW,), jnp.float32),
        pltpu.VMEM_SHARED((16, SROW), jnp.float32),
        pltpu.VMEM((TPB, SROW), jnp.float32),
        pltpu.VMEM((PREC,), jnp.float32),
        pltpu.VMEM((NI * EE,), jnp.float32),
        pltpu.VMEM((OROWS * 16,), jnp.float32),
        pltpu.SemaphoreType.DMA,
        pltpu.SemaphoreType.DMA,
    ],
    compiler_params=pltpu.CompilerParams(needs_layout_passes=False),
)
def _main(
    emb, mask, out,
    embbuf, maskbuf, lsums, lcnt, pbuf, shared, mates, macc, meansbuf, cbuf,
    sem0, sem1,
):
    cid = lax.axis_index("c")
    sid = lax.axis_index("s")
    b = cid * 2 + sid // 8
    tile_base = (sid % 8) * PIX_PER_TILE
    orow = cid * 16 + sid

    iota = lax.iota(jnp.int32, 16)
    lane_s = iota * LSTRIDE
    lane_c = iota * NI
    ones = jnp.ones((16,), jnp.float32)
    zf = jnp.zeros((16,), jnp.float32)

    _zero_ref(lsums, 16 * LSTRIDE - 16)
    lsums[pl.ds(16 * LSTRIDE - 16, 16)] = zf
    _zero_ref(lcnt, 16 * NI)

    def p1(slot, mv):
        def body(g, mvi):
            off = g * 16
            m = maskbuf[slot, pl.ds(off, 16)]
            plsc.addupdate_scatter(lcnt, [lane_c + m], ones)
            basei = lane_s + (m << 4)
            for c in range(EE):
                v = embbuf[slot, c, pl.ds(off, 16)]
                plsc.addupdate_scatter(lsums, [basei + c], v)
            return jnp.maximum(mvi, m)

        return plsc.parallel_loop(0, NGROUP, 1, unroll=2, carry=mv)(body)

    maxv = _pixel_pipeline(
        emb, mask, b, tile_base, embbuf, maskbuf, (sem0, sem1), p1,
        jnp.zeros((16,), jnp.int32),
    )

    def sum_row(m, _):
        acc = zf
        for l in range(16):
            acc = acc + plsc.load_gather(lsums, [iota + (l * LSTRIDE) + m * EE])
        pbuf[pl.ds(m * 16, 16)] = acc
        return 0

    lax.fori_loop(0, NI, sum_row, 0)
    comp = zf
    for l in range(16):
        comp = comp + plsc.load_gather(lcnt, [iota + l * NI])
    pbuf[pl.ds(NI * EE, 16)] = comp
    v16 = plsc.load_gather(lcnt, [lane_c + 16])
    pbuf[pl.ds(NI * EE + 16, 16)] = jnp.full((16,), jnp.sum(v16), jnp.float32)
    nmax = jnp.max(maxv).astype(jnp.float32)
    pbuf[pl.ds(NI * EE + 32, 16)] = jnp.full((16,), nmax, jnp.float32)

    pltpu.sync_copy(pbuf, shared.at[sid])
    plsc.subcore_barrier()
    g0 = (sid // 8) * TPB
    for t in range(TPB):
        pltpu.sync_copy(shared.at[g0 + t], mates.at[t])

    def mred(j, _):
        off = j * 16
        a = mates[0, pl.ds(off, 16)]
        for t in range(1, TPB):
            a = a + mates[t, pl.ds(off, 16)]
        macc[pl.ds(off, 16)] = a
        return 0

    lax.fori_loop(0, PREC // 16 - 1, mred, 0)
    a = mates[0, pl.ds(PREC - 16, 16)]
    for t in range(1, TPB):
        a = jnp.maximum(a, mates[t, pl.ds(PREC - 16, 16)])
    macc[pl.ds(PREC - 16, 16)] = a

    def mean_row(m, _):
        cvec = plsc.load_gather(macc, [jnp.full((16,), NI * EE, jnp.int32) + m])
        row = macc[pl.ds(m * 16, 16)]
        meansbuf[pl.ds(m * 16, 16)] = jnp.where(
            cvec > 0.0, row / jnp.maximum(cvec, 1.0), 0.0
        )
        return 0

    lax.fori_loop(0, NI, mean_row, 0)

    _zero_ref(lcnt, 16 * NI)

    def p2(slot, cy):
        def body(g):
            off = g * 16
            m = maskbuf[slot, pl.ds(off, 16)]
            base16 = m << 4
            accs = [zf for _ in range(4)]
            for c in range(EE):
                v = embbuf[slot, c, pl.ds(off, 16)]
                mu = plsc.load_gather(meansbuf, [base16 + c])
                dlt = v - mu
                accs[c % 4] = accs[c % 4] + dlt * dlt
            acc = (accs[0] + accs[1]) + (accs[2] + accs[3])
            dist = _vsqrt(acc)
            h = jnp.maximum(dist - DELTA_V, 0.0)
            plsc.addupdate_scatter(lcnt, [lane_c + m], h * h)

        plsc.parallel_loop(0, NGROUP, 1, unroll=2)(body)
        return cy

    _pixel_pipeline(emb, mask, b, tile_base, embbuf, maskbuf, (sem0, sem1), p2, 0)

    def var_row(m, _):
        v = plsc.load_gather(lcnt, [lane_c + m])
        cbuf[pl.ds(m * 16, 16)] = jnp.full((16,), jnp.sum(v), jnp.float32)
        return 0

    lax.fori_loop(0, NI, var_row, 0)

    nvec = macc[pl.ds(PREC - 16, 16)]
    idsf = (iota + 1).astype(jnp.float32)
    mjs = tuple(
        plsc.load_gather(meansbuf, [(iota + 1) * 16 + c]) for c in range(EE)
    )
    validj = jnp.where(idsf <= nvec, 1.0, 0.0)

    def pair_i(i, carry):
        psum, pcnt = carry
        base = (i + 1) * 16
        accs = [zf for _ in range(4)]
        for c in range(EE):
            mi = plsc.load_gather(meansbuf, [jnp.full((16,), base, jnp.int32) + c])
            d = mjs[c] - mi
            accs[c % 4] = accs[c % 4] + d * d
        acc = (accs[0] + accs[1]) + (accs[2] + accs[3])
        dist = _vsqrt(acc)
        h = jnp.maximum(2.0 * DELTA_D - dist, 0.0)
        ifl = (i + 1).astype(jnp.float32)
        pv = jnp.where((idsf > ifl) & (ifl <= nvec), validj, 0.0)
        return psum + h * h * pv, pcnt + pv

    psum, pcnt = lax.fori_loop(0, 16, pair_i, (zf, zf))
    tot = jnp.full((16,), jnp.sum(psum), jnp.float32)
    npair = jnp.full((16,), jnp.sum(pcnt), jnp.float32)
    dist_row = jnp.where(nvec > 1.0, tot / jnp.maximum(npair, 1.0), 0.0)
    cbuf[pl.ds(NI * 16, 16)] = dist_row

    nsq = zf
    for c in range(EE):
        nsq = nsq + mjs[c] * mjs[c]
    nr = _vsqrt(nsq)
    regsum = jnp.full((16,), jnp.sum(nr * validj), jnp.float32)
    reg_row = jnp.where(nvec > 0.0, regsum / jnp.maximum(nvec, 1.0), 0.0)
    cbuf[pl.ds((NI + 1) * 16, 16)] = reg_row

    def cnt_row(m, _):
        cvec = plsc.load_gather(macc, [jnp.full((16,), NI * EE, jnp.int32) + m])
        cbuf[pl.ds((NI + 2) * 16 + m * 16, 16)] = cvec
        return 0

    lax.fori_loop(0, NI, cnt_row, 0)
    cbuf[pl.ds((2 * NI + 2) * 16, 16)] = nvec
    for r in range(2 * NI + 3, OROWS):
        cbuf[pl.ds(r * 16, 16)] = zf

    pltpu.sync_copy(cbuf, out.at[orow])


def _fin_body(v_ref, out_ref):
    tv = jnp.float32(0.0)
    td = jnp.float32(0.0)
    tr = jnp.float32(0.0)
    valid = jnp.float32(0.0)
    for b in range(BB):
        V = v_ref[TPB * b]
        for t in range(1, TPB):
            V = V + v_ref[TPB * b + t]
        Vm = V[1:NI]
        lead = v_ref[TPB * b]
        C = lead[NI + 3 : 2 * NI + 2]
        n_rep = lead[2 * NI + 2 : 2 * NI + 3]
        lm = Vm / jnp.maximum(C, 1.0)
        pres = jnp.where(C > 0.0, 1.0, 0.0)
        npres = jnp.sum(pres[:, 0:1])
        vsum = jnp.sum(lm[:, 0:1])
        v_b = jnp.where(npres > 0.0, vsum / jnp.maximum(npres, 1.0), 0.0)
        n_sc = jnp.max(n_rep)
        has = jnp.where(n_sc > 0.0, 1.0, 0.0)
        tv = tv + has * v_b
        td = td + has * jnp.max(lead[NI : NI + 1])
        tr = tr + has * jnp.max(lead[NI + 1 : NI + 2])
        valid = valid + has
    vf = jnp.maximum(valid, 1.0)
    tv = jnp.where(valid > 0.0, tv / vf, tv)
    td = jnp.where(valid > 0.0, td / vf, td)
    tr = jnp.where(valid > 0.0, tr / vf, tr)
    loss = ALPHA * tv + BETA * td + GAMMA * tr
    out_ref[0:1, :] = jnp.full((1, 128), loss, jnp.float32)
    out_ref[1:2, :] = jnp.full((1, 128), tv, jnp.float32)
    out_ref[2:3, :] = jnp.full((1, 128), td, jnp.float32)
    out_ref[3:4, :] = jnp.full((1, 128), tr, jnp.float32)
    out_ref[4:8, :] = jnp.zeros((4, 128), jnp.float32)


_fin = pl.pallas_call(
    _fin_body,
    out_shape=jax.ShapeDtypeStruct((8, 128), jnp.float32),
)


@jax.jit
def kernel(embeddings, instance_masks):
    emb = embeddings.reshape(BB, EE, PP)
    mask = instance_masks.reshape(BB, PP).astype(jnp.int32)
    part = _main(emb, mask)
    out = _fin(part.reshape(NW, OROWS, 16))
    return out[0, 0], out[1, 0], out[2, 0], out[3, 0]


# speedup vs baseline: 1.1808x; 1.1247x over previous
import functools

import jax
import jax.numpy as jnp
from jax import lax
from jax.experimental import pallas as pl
from jax.experimental.pallas import tpu as pltpu
from jax.experimental.pallas import tpu_sc as plsc

DELTA_V = 0.5
DELTA_D = 1.5
ALPHA = 1.0
BETA = 1.0
GAMMA = 0.001

BB = 4
EE = 16
PP = 512 * 512
NI = 17

NW = 32
TPB = NW // BB
PIX_PER_TILE = PP // TPB
CHUNK = 2048
NGROUP = CHUNK // 16
NCHUNK = PIX_PER_TILE // CHUNK

LSTRIDE = NI * EE + 1

PREC = NI * EE + 3 * 16
SROW = 512
OROWS = 40

_mesh = plsc.VectorSubcoreMesh(core_axis_name="c", subcore_axis_name="s")


def _vsqrt(x):
    i = lax.bitcast_convert_type(x, jnp.int32)
    r = lax.bitcast_convert_type(jnp.int32(0x5F3759DF) - (i >> 1), jnp.float32)
    for _ in range(3):
        r = r * (1.5 - 0.5 * x * r * r)
    return x * r


def _zero_ref(ref, nwords):
    zf = jnp.zeros((16,), jnp.float32)

    def body(j, _):
        ref[pl.ds(j * 16, 16)] = zf
        return 0

    lax.fori_loop(0, nwords // 16, body, 0)


def _pixel_pipeline(emb, mask, b, tile_base, embbuf, maskbuf, sems, process, carry0):
    def start(slot, cbase):
        pltpu.async_copy(
            emb.at[b, :, pl.ds(cbase, CHUNK)], embbuf.at[slot], sems[slot]
        )
        pltpu.async_copy(mask.at[b, pl.ds(cbase, CHUNK)], maskbuf.at[slot], sems[slot])

    def drain(slot):
        pltpu.make_async_copy(
            emb.at[0, :, pl.ds(0, CHUNK)], embbuf.at[slot], sems[slot]
        ).wait()
        pltpu.make_async_copy(
            mask.at[0, pl.ds(0, CHUNK)], maskbuf.at[slot], sems[slot]
        ).wait()

    start(0, tile_base)

    def pair(k2, cy):
        base0 = tile_base + (2 * k2) * CHUNK
        start(1, base0 + CHUNK)
        drain(0)
        cy = process(0, cy)
        start(0, jnp.minimum(base0 + 2 * CHUNK, PP - CHUNK))
        drain(1)
        cy = process(1, cy)
        return cy

    carry = lax.fori_loop(0, NCHUNK // 2, pair, carry0)
    drain(0)
    return carry


@functools.partial(
    pl.kernel,
    out_type=jax.ShapeDtypeStruct((NW, OROWS * 16), jnp.float32),
    mesh=_mesh,
    scratch_types=[
        pltpu.VMEM((2, EE, CHUNK), jnp.float32),
        pltpu.VMEM((2, CHUNK), jnp.int32),
        pltpu.VMEM((16 * LSTRIDE,), jnp.float32),
        pltpu.VMEM((16 * NI,), jnp.float32),
        pltpu.VMEM((SROW,), jnp.float32),
        pltpu.VMEM_SHARED((16, SROW), jnp.float32),
        pltpu.VMEM((TPB, SROW), jnp.float32),
        pltpu.VMEM((PREC,), jnp.float32),
        pltpu.VMEM((NI * EE,), jnp.float32),
        pltpu.VMEM((OROWS * 16,), jnp.float32),
        pltpu.SemaphoreType.DMA,
        pltpu.SemaphoreType.DMA,
    ],
    compiler_params=pltpu.CompilerParams(needs_layout_passes=False),
)
def _main(
    emb, mask, out,
    embbuf, maskbuf, lsums, lcnt, pbuf, shared, mates, macc, meansbuf, cbuf,
    sem0, sem1,
):
    cid = lax.axis_index("c")
    sid = lax.axis_index("s")
    b = cid * 2 + sid // 8
    tile_base = (sid % 8) * PIX_PER_TILE
    orow = cid * 16 + sid

    iota = lax.iota(jnp.int32, 16)
    lane_s = iota * LSTRIDE
    lane_c = iota * NI
    ones = jnp.ones((16,), jnp.float32)
    zf = jnp.zeros((16,), jnp.float32)

    _zero_ref(lsums, 16 * LSTRIDE - 16)
    lsums[pl.ds(16 * LSTRIDE - 16, 16)] = zf
    _zero_ref(lcnt, 16 * NI)

    def p1(slot, mv):
        def body(g, mvi):
            off = g * 16
            m = maskbuf[slot, pl.ds(off, 16)]
            plsc.addupdate_scatter(lcnt, [lane_c + m], ones)
            basei = lane_s + (m << 4)
            for c in range(EE):
                v = embbuf[slot, c, pl.ds(off, 16)]
                plsc.addupdate_scatter(lsums, [basei + c], v)
            return jnp.maximum(mvi, m)

        return plsc.parallel_loop(0, NGROUP, 1, unroll=2, carry=mv)(body)

    maxv = _pixel_pipeline(
        emb, mask, b, tile_base, embbuf, maskbuf, (sem0, sem1), p1,
        jnp.zeros((16,), jnp.int32),
    )

    def sum_row(m, _):
        acc = zf
        for l in range(16):
            acc = acc + plsc.load_gather(lsums, [iota + (l * LSTRIDE) + m * EE])
        pbuf[pl.ds(m * 16, 16)] = acc
        return 0

    lax.fori_loop(0, NI, sum_row, 0)
    comp = zf
    for l in range(16):
        comp = comp + plsc.load_gather(lcnt, [iota + l * NI])
    pbuf[pl.ds(NI * EE, 16)] = comp
    v16 = plsc.load_gather(lcnt, [lane_c + 16])
    pbuf[pl.ds(NI * EE + 16, 16)] = jnp.full((16,), jnp.sum(v16), jnp.float32)
    nmax = jnp.max(maxv).astype(jnp.float32)
    pbuf[pl.ds(NI * EE + 32, 16)] = jnp.full((16,), nmax, jnp.float32)

    pltpu.sync_copy(pbuf, shared.at[sid])
    plsc.subcore_barrier()
    g0 = (sid // 8) * TPB
    for t in range(TPB):
        pltpu.sync_copy(shared.at[g0 + t], mates.at[t])

    def mred(j, _):
        off = j * 16
        a = mates[0, pl.ds(off, 16)]
        for t in range(1, TPB):
            a = a + mates[t, pl.ds(off, 16)]
        macc[pl.ds(off, 16)] = a
        return 0

    lax.fori_loop(0, PREC // 16 - 1, mred, 0)
    a = mates[0, pl.ds(PREC - 16, 16)]
    for t in range(1, TPB):
        a = jnp.maximum(a, mates[t, pl.ds(PREC - 16, 16)])
    macc[pl.ds(PREC - 16, 16)] = a

    def mean_row(m, _):
        cvec = plsc.load_gather(macc, [jnp.full((16,), NI * EE, jnp.int32) + m])
        row = macc[pl.ds(m * 16, 16)]
        meansbuf[pl.ds(m * 16, 16)] = jnp.where(
            cvec > 0.0, row / jnp.maximum(cvec, 1.0), 0.0
        )
        return 0

    lax.fori_loop(0, NI, mean_row, 0)

    _zero_ref(lcnt, 16 * NI)

    def p2(slot, cy):
        def body(g):
            off = g * 16
            m = maskbuf[slot, pl.ds(off, 16)]
            base16 = m << 4
            accs = [zf for _ in range(4)]
            for c in range(EE):
                v = embbuf[slot, c, pl.ds(off, 16)]
                mu = plsc.load_gather(meansbuf, [base16 + c])
                dlt = v - mu
                accs[c % 4] = accs[c % 4] + dlt * dlt
            acc = (accs[0] + accs[1]) + (accs[2] + accs[3])
            dist = _vsqrt(acc)
            h = jnp.maximum(dist - DELTA_V, 0.0)
            plsc.addupdate_scatter(lcnt, [lane_c + m], h * h)

        plsc.parallel_loop(0, NGROUP, 1, unroll=2)(body)
        return cy

    _pixel_pipeline(emb, mask, b, tile_base, embbuf, maskbuf, (sem0, sem1), p2, 0)

    def var_row(m, _):
        v = plsc.load_gather(lcnt, [lane_c + m])
        cbuf[pl.ds(m * 16, 16)] = jnp.full((16,), jnp.sum(v), jnp.float32)
        return 0

    lax.fori_loop(0, NI, var_row, 0)

    nvec = macc[pl.ds(PREC - 16, 16)]
    idsf = (iota + 1).astype(jnp.float32)
    mjs = tuple(
        plsc.load_gather(meansbuf, [(iota + 1) * 16 + c]) for c in range(EE)
    )
    validj = jnp.where(idsf <= nvec, 1.0, 0.0)

    def pair_i(i, carry):
        psum, pcnt = carry
        base = (i + 1) * 16
        accs = [zf for _ in range(4)]
        for c in range(EE):
            mi = plsc.load_gather(meansbuf, [jnp.full((16,), base, jnp.int32) + c])
            d = mjs[c] - mi
            accs[c % 4] = accs[c % 4] + d * d
        acc = (accs[0] + accs[1]) + (accs[2] + accs[3])
        dist = _vsqrt(acc)
        h = jnp.maximum(2.0 * DELTA_D - dist, 0.0)
        ifl = (i + 1).astype(jnp.float32)
        pv = jnp.where((idsf > ifl) & (ifl <= nvec), validj, 0.0)
        return psum + h * h * pv, pcnt + pv

    psum, pcnt = lax.fori_loop(0, 16, pair_i, (zf, zf))
    tot = jnp.full((16,), jnp.sum(psum), jnp.float32)
    npair = jnp.full((16,), jnp.sum(pcnt), jnp.float32)
    dist_row = jnp.where(nvec > 1.0, tot / jnp.maximum(npair, 1.0), 0.0)
    cbuf[pl.ds(NI * 16, 16)] = dist_row

    nsq = zf
    for c in range(EE):
        nsq = nsq + mjs[c] * mjs[c]
    nr = _vsqrt(nsq)
    regsum = jnp.full((16,), jnp.sum(nr * validj), jnp.float32)
    reg_row = jnp.where(nvec > 0.0, regsum / jnp.maximum(nvec, 1.0), 0.0)
    cbuf[pl.ds((NI + 1) * 16, 16)] = reg_row

    def cnt_row(m, _):
        cvec = plsc.load_gather(macc, [jnp.full((16,), NI * EE, jnp.int32) + m])
        cbuf[pl.ds((NI + 2) * 16 + m * 16, 16)] = cvec
        return 0

    lax.fori_loop(0, NI, cnt_row, 0)
    cbuf[pl.ds((2 * NI + 2) * 16, 16)] = nvec
    for r in range(2 * NI + 3, OROWS):
        cbuf[pl.ds(r * 16, 16)] = zf

    pltpu.sync_copy(cbuf, out.at[orow])


def _fin_body(v_ref, out_ref):
    tv = jnp.float32(0.0)
    td = jnp.float32(0.0)
    tr = jnp.float32(0.0)
    valid = jnp.float32(0.0)
    for b in range(BB):
        V = v_ref[TPB * b]
        for t in range(1, TPB):
            V = V + v_ref[TPB * b + t]
        Vm = V[1:NI]
        lead = v_ref[TPB * b]
        C = lead[NI + 3 : 2 * NI + 2]
        n_rep = lead[2 * NI + 2 : 2 * NI + 3]
        lm = Vm / jnp.maximum(C, 1.0)
        pres = jnp.where(C > 0.0, 1.0, 0.0)
        npres = jnp.sum(pres[:, 0:1])
        vsum = jnp.sum(lm[:, 0:1])
        v_b = jnp.where(npres > 0.0, vsum / jnp.maximum(npres, 1.0), 0.0)
        n_sc = jnp.max(n_rep)
        has = jnp.where(n_sc > 0.0, 1.0, 0.0)
        tv = tv + has * v_b
        td = td + has * jnp.max(lead[NI : NI + 1])
        tr = tr + has * jnp.max(lead[NI + 1 : NI + 2])
        valid = valid + has
    vf = jnp.maximum(valid, 1.0)
    tv = jnp.where(valid > 0.0, tv / vf, tv)
    td = jnp.where(valid > 0.0, td / vf, td)
    tr = jnp.where(valid > 0.0, tr / vf, tr)
    loss = ALPHA * tv + BETA * td + GAMMA * tr
    out_ref[0:1, :] = jnp.full((1, 128), loss, jnp.float32)
    out_ref[1:2, :] = jnp.full((1, 128), tv, jnp.float32)
    out_ref[2:3, :] = jnp.full((1, 128), td, jnp.float32)
    out_ref[3:4, :] = jnp.full((1, 128), tr, jnp.float32)
    out_ref[4:8, :] = jnp.zeros((4, 128), jnp.float32)


_fin = pl.pallas_call(
    _fin_body,
    out_shape=jax.ShapeDtypeStruct((8, 128), jnp.float32),
)


@jax.jit
def kernel(embeddings, instance_masks):
    emb = embeddings.reshape(BB, EE, PP)
    mask = instance_masks.reshape(BB, PP).astype(jnp.int32)
    part = _main(emb, mask)
    out = _fin(part.reshape(NW, OROWS, 16))
    return out[0, 0], out[1, 0], out[2, 0], out[3, 0]
